# baseline (speedup 1.0000x reference)
import jax
import jax.numpy as jnp
from jax.experimental import pallas as pl
from jax.experimental.pallas import tpu as pltpu

B, J, V = 32, 8, 32768
VC = 2048
NCHUNK = V // VC


def _fused_body(a_ref, e_ref, logits_ref, stddev_ref, best_ref,
                macc_ref, cidx_ref):
    j = pl.program_id(0)

    a = a_ref[...]
    e = e_ref[...]

    m = jnp.min(a, axis=1)
    iota_j = jax.lax.broadcasted_iota(jnp.int32, (B, J, VC), 1)
    jsel = jnp.min(jnp.where(a == m[:, None, :], iota_j, J), axis=1)
    s = jnp.sum(jnp.where(iota_j == jsel[:, None, :], e, 0.0), axis=1)
    logits_ref[...] = m
    stddev_ref[...] = s

    cm = jnp.max(a, axis=2)
    iota_v = jax.lax.broadcasted_iota(jnp.int32, (B, J, VC), 2)
    li = jnp.min(jnp.where(a == cm[:, :, None], iota_v, V), axis=2) + j * VC
    upd = (cm > macc_ref[...]) | (j == 0)
    macc_ref[...] = jnp.where(upd, cm, macc_ref[...])
    cidx_ref[...] = jnp.where(upd, li, cidx_ref[...])
    best_ref[...] = cidx_ref[...]


@jax.jit
def kernel(action, explore_rate):
    logits, stddev, best2d = pl.pallas_call(
        _fused_body,
        grid=(NCHUNK,),
        in_specs=[
            pl.BlockSpec((B, J, VC), lambda j: (0, 0, j)),
            pl.BlockSpec((B, J, VC), lambda j: (0, 0, j)),
        ],
        out_specs=[
            pl.BlockSpec((B, VC), lambda j: (0, j)),
            pl.BlockSpec((B, VC), lambda j: (0, j)),
            pl.BlockSpec((B, J), lambda j: (0, 0)),
        ],
        out_shape=[
            jax.ShapeDtypeStruct((B, V), jnp.float32),
            jax.ShapeDtypeStruct((B, V), jnp.float32),
            jax.ShapeDtypeStruct((B, J), jnp.int32),
        ],
        scratch_shapes=[
            pltpu.VMEM((B, J), jnp.float32),
            pltpu.VMEM((B, J), jnp.int32),
        ],
        compiler_params=pltpu.CompilerParams(
            dimension_semantics=("arbitrary",),
        ),
    )(action, explore_rate)
    return logits, stddev, best2d[:, None, :]


# speedup vs baseline: 1.0112x; 1.0112x over previous
import jax
import jax.numpy as jnp
from jax.experimental import pallas as pl
from jax.experimental.pallas import tpu as pltpu

B, J, V = 32, 8, 32768
VC = 2048
NCHUNK = V // VC


def _fused_body(a_ref, e_ref, logits_ref, stddev_ref, best_ref,
                macc_ref, cidx_ref):
    j = pl.program_id(0)

    a = a_ref[...]
    e = e_ref[...]

    m = jnp.min(a, axis=1)
    iota_j = jax.lax.broadcasted_iota(jnp.int32, (B, J, VC), 1)
    jsel = jnp.min(jnp.where(a == m[:, None, :], iota_j, J), axis=1)
    s = jnp.sum(jnp.where(iota_j == jsel[:, None, :], e, 0.0), axis=1)
    logits_ref[...] = m
    stddev_ref[...] = s

    upd = (a > macc_ref[...]) | (j == 0)
    macc_ref[...] = jnp.where(upd, a, macc_ref[...])
    cidx_ref[...] = jnp.where(upd, j, cidx_ref[...])

    @pl.when(j == NCHUNK - 1)
    def _():
        macc = macc_ref[...]
        cm = jnp.max(macc, axis=2)
        lane = jax.lax.broadcasted_iota(jnp.int32, (B, J, VC), 2)
        gidx = cidx_ref[...] * VC + lane
        best_ref[...] = jnp.min(
            jnp.where(macc == cm[:, :, None], gidx, V), axis=2)


@jax.jit
def kernel(action, explore_rate):
    logits, stddev, best2d = pl.pallas_call(
        _fused_body,
        grid=(NCHUNK,),
        in_specs=[
            pl.BlockSpec((B, J, VC), lambda j: (0, 0, j)),
            pl.BlockSpec((B, J, VC), lambda j: (0, 0, j)),
        ],
        out_specs=[
            pl.BlockSpec((B, VC), lambda j: (0, j)),
            pl.BlockSpec((B, VC), lambda j: (0, j)),
            pl.BlockSpec((B, J), lambda j: (0, 0)),
        ],
        out_shape=[
            jax.ShapeDtypeStruct((B, V), jnp.float32),
            jax.ShapeDtypeStruct((B, V), jnp.float32),
            jax.ShapeDtypeStruct((B, J), jnp.int32),
        ],
        scratch_shapes=[
            pltpu.VMEM((B, J, VC), jnp.float32),
            pltpu.VMEM((B, J, VC), jnp.int32),
        ],
        compiler_params=pltpu.CompilerParams(
            dimension_semantics=("arbitrary",),
        ),
    )(action, explore_rate)
    return logits, stddev, best2d[:, None, :]
